# baseline (speedup 1.0000x reference)
import functools

import jax
import jax.numpy as jnp
from jax import lax
from jax.experimental import pallas as pl
from jax.experimental.pallas import tpu as pltpu
from jax.experimental.pallas import tpu_sc as plsc

N_ROWS = 1000000
EMB_DIM = 32
LANES = 16


def _make_sc_kernel(batch):
    info = plsc.get_sparse_core_info()
    nc, ns = info.num_cores, info.num_subcores
    nw = nc * ns
    assert batch % (8 * nw) == 0
    b_per_w = batch // nw
    n_groups = b_per_w // LANES
    mesh = plsc.VectorSubcoreMesh(core_axis_name="c", subcore_axis_name="s")

    @functools.partial(
        pl.kernel,
        mesh=mesh,
        out_type=jax.ShapeDtypeStruct((batch,), jnp.float32),
        scratch_types=[
            pltpu.VMEM((b_per_w,), jnp.int32),
            pltpu.VMEM((b_per_w,), jnp.int32),
            pltpu.VMEM((EMB_DIM * b_per_w,), jnp.float32),
            pltpu.VMEM((EMB_DIM * b_per_w,), jnp.float32),
            pltpu.VMEM((b_per_w,), jnp.float32),
            pltpu.VMEM((b_per_w,), jnp.float32),
            pltpu.VMEM((LANES,), jnp.float32),
            pltpu.VMEM((b_per_w,), jnp.float32),
            pltpu.SemaphoreType.DMA,
        ],
    )
    def k(users_hbm, items_hbm, uflat_hbm, iflat_hbm, ubias_hbm, ibias_hbm,
          bias_hbm, out_hbm, users_v, items_v, udat_v, idat_v,
          ubias_v, ibias_v, bias_v, out_v, sem):
        wid = lax.axis_index("s") * nc + lax.axis_index("c")
        base = wid * b_per_w

        pltpu.sync_copy(users_hbm.at[pl.ds(base, b_per_w)], users_v)
        pltpu.sync_copy(items_hbm.at[pl.ds(base, b_per_w)], items_v)
        pltpu.sync_copy(bias_hbm, bias_v)

        copies = []
        for d in range(EMB_DIM):
            cp = pltpu.make_async_copy(
                uflat_hbm.at[pl.ds(d * N_ROWS, N_ROWS)].at[users_v],
                udat_v.at[pl.ds(d * b_per_w, b_per_w)], sem)
            cp.start()
            copies.append(cp)
            cp = pltpu.make_async_copy(
                iflat_hbm.at[pl.ds(d * N_ROWS, N_ROWS)].at[items_v],
                idat_v.at[pl.ds(d * b_per_w, b_per_w)], sem)
            cp.start()
            copies.append(cp)
        cp_ub = pltpu.make_async_copy(ubias_hbm.at[users_v], ubias_v, sem)
        cp_ib = pltpu.make_async_copy(ibias_hbm.at[items_v], ibias_v, sem)
        cp_ub.start()
        cp_ib.start()
        for cp in copies:
            cp.wait()
        cp_ub.wait()
        cp_ib.wait()

        bias_vec = bias_v[...]

        def group(g, carry):
            e0 = g * LANES
            acc = jnp.zeros((LANES,), jnp.float32)
            for d in range(EMB_DIM):
                sl = pl.ds(d * b_per_w + e0, LANES)
                acc = acc + udat_v[sl] * idat_v[sl]
            sl = pl.ds(e0, LANES)
            acc = acc + ubias_v[sl] + ibias_v[sl] + bias_vec
            out_v[sl] = 1.0 / (1.0 + jnp.exp(-acc))
            return carry

        lax.fori_loop(0, n_groups, group, 0)
        pltpu.sync_copy(out_v, out_hbm.at[pl.ds(base, b_per_w)])

    return k


@jax.jit
def kernel(users, items, user_table, item_table, user_bias, item_bias, bias):
    batch = users.shape[0]
    users = users.astype(jnp.int32)
    items = items.astype(jnp.int32)
    uflat = user_table.T.reshape(-1)
    iflat = item_table.T.reshape(-1)
    bias16 = jnp.broadcast_to(bias.astype(jnp.float32), (LANES,))
    k = _make_sc_kernel(batch)
    return k(users, items, uflat, iflat, user_bias, item_bias, bias16)


# speedup vs baseline: 20.8409x; 20.8409x over previous
import functools

import jax
import jax.numpy as jnp
from jax import lax
from jax.experimental import pallas as pl
from jax.experimental.pallas import tpu as pltpu
from jax.experimental.pallas import tpu_sc as plsc

N_ROWS = 1000000
EMB_DIM = 32
LANES = 16
PANELS = 8192
DIM_STRIDE = PANELS * 128
BLK_COLS = 8192


def _detile_body(u_in, i_in, u_out, i_out):
    u_out[...] = u_in[...].reshape(EMB_DIM, BLK_COLS // 128, 128)
    i_out[...] = i_in[...].reshape(EMB_DIM, BLK_COLS // 128, 128)


def _detile(ut, it):
    grid = (N_ROWS + BLK_COLS - 1) // BLK_COLS
    out_shape = jax.ShapeDtypeStruct((EMB_DIM, PANELS, 128), jnp.float32)
    in_spec = pl.BlockSpec((EMB_DIM, BLK_COLS), lambda j: (0, j))
    out_spec = pl.BlockSpec((EMB_DIM, BLK_COLS // 128, 128),
                            lambda j: (0, j, 0))
    return pl.pallas_call(
        _detile_body,
        grid=(grid,),
        in_specs=[in_spec, in_spec],
        out_specs=[out_spec, out_spec],
        out_shape=[out_shape, out_shape],
    )(ut, it)


def _make_sc_kernel(batch):
    info = plsc.get_sparse_core_info()
    nc, ns = info.num_cores, info.num_subcores
    nw = nc * ns
    assert batch % (8 * nw) == 0
    b_per_w = batch // nw
    n_groups = b_per_w // LANES
    mesh = plsc.VectorSubcoreMesh(core_axis_name="c", subcore_axis_name="s")

    @functools.partial(
        pl.kernel,
        mesh=mesh,
        out_type=jax.ShapeDtypeStruct((batch,), jnp.float32),
        scratch_types=[
            pltpu.VMEM((b_per_w,), jnp.int32),
            pltpu.VMEM((b_per_w,), jnp.int32),
            pltpu.VMEM((EMB_DIM * b_per_w,), jnp.float32),
            pltpu.VMEM((EMB_DIM * b_per_w,), jnp.float32),
            pltpu.VMEM((b_per_w,), jnp.float32),
            pltpu.VMEM((b_per_w,), jnp.float32),
            pltpu.VMEM((LANES,), jnp.float32),
            pltpu.VMEM((b_per_w,), jnp.float32),
            pltpu.SemaphoreType.DMA,
        ],
    )
    def k(users_hbm, items_hbm, uflat_hbm, iflat_hbm, ubias_hbm, ibias_hbm,
          bias_hbm, out_hbm, users_v, items_v, udat_v, idat_v,
          ubias_v, ibias_v, bias_v, out_v, sem):
        wid = lax.axis_index("s") * nc + lax.axis_index("c")
        base = wid * b_per_w

        pltpu.sync_copy(users_hbm.at[pl.ds(base, b_per_w)], users_v)
        pltpu.sync_copy(items_hbm.at[pl.ds(base, b_per_w)], items_v)
        pltpu.sync_copy(bias_hbm, bias_v)

        copies = []
        for d in range(EMB_DIM):
            cp = pltpu.make_async_copy(
                uflat_hbm.at[pl.ds(d * DIM_STRIDE, DIM_STRIDE)].at[users_v],
                udat_v.at[pl.ds(d * b_per_w, b_per_w)], sem)
            cp.start()
            copies.append(cp)
            cp = pltpu.make_async_copy(
                iflat_hbm.at[pl.ds(d * DIM_STRIDE, DIM_STRIDE)].at[items_v],
                idat_v.at[pl.ds(d * b_per_w, b_per_w)], sem)
            cp.start()
            copies.append(cp)
        cp_ub = pltpu.make_async_copy(ubias_hbm.at[users_v], ubias_v, sem)
        cp_ib = pltpu.make_async_copy(ibias_hbm.at[items_v], ibias_v, sem)
        cp_ub.start()
        cp_ib.start()
        for cp in copies:
            cp.wait()
        cp_ub.wait()
        cp_ib.wait()

        bias_vec = bias_v[...]

        def group(g, carry):
            e0 = g * LANES
            acc = jnp.zeros((LANES,), jnp.float32)
            for d in range(EMB_DIM):
                sl = pl.ds(d * b_per_w + e0, LANES)
                acc = acc + udat_v[sl] * idat_v[sl]
            sl = pl.ds(e0, LANES)
            acc = acc + ubias_v[sl] + ibias_v[sl] + bias_vec
            out_v[sl] = 1.0 / (1.0 + jnp.exp(-acc))
            return carry

        lax.fori_loop(0, n_groups, group, 0)
        pltpu.sync_copy(out_v, out_hbm.at[pl.ds(base, b_per_w)])

    return k


@jax.jit
def kernel(users, items, user_table, item_table, user_bias, item_bias, bias):
    batch = users.shape[0]
    users = users.astype(jnp.int32)
    items = items.astype(jnp.int32)
    u3, i3 = _detile(user_table.T, item_table.T)
    uflat = u3.reshape(-1)
    iflat = i3.reshape(-1)
    bias16 = jnp.broadcast_to(bias.astype(jnp.float32), (LANES,))
    k = _make_sc_kernel(batch)
    return k(users, items, uflat, iflat, user_bias, item_bias, bias16)


# speedup vs baseline: 25.0835x; 1.2036x over previous
import functools

import jax
import jax.numpy as jnp
from jax import lax
from jax.experimental import pallas as pl
from jax.experimental.pallas import tpu as pltpu
from jax.experimental.pallas import tpu_sc as plsc

N_ROWS = 1000000
EMB_DIM = 32
PAIRS = EMB_DIM // 2
LANES = 16
PANELS = 8192
DIM_STRIDE = PANELS * 128
BLK_COLS = 8192


def _pack_block(x):
    lo = jax.lax.bitcast_convert_type(x[:PAIRS, :].astype(jnp.bfloat16),
                                      jnp.uint16).astype(jnp.uint32)
    hi = jax.lax.bitcast_convert_type(x[PAIRS:, :].astype(jnp.bfloat16),
                                      jnp.uint16).astype(jnp.uint32)
    packed = lo | (hi << 16)
    return packed.reshape(PAIRS, BLK_COLS // 128, 128)


def _detile_body(u_in, i_in, u_out, i_out):
    u_out[...] = _pack_block(u_in[...])
    i_out[...] = _pack_block(i_in[...])


def _detile(ut, it):
    grid = (N_ROWS + BLK_COLS - 1) // BLK_COLS
    out_shape = jax.ShapeDtypeStruct((PAIRS, PANELS, 128), jnp.uint32)
    in_spec = pl.BlockSpec((EMB_DIM, BLK_COLS), lambda j: (0, j))
    out_spec = pl.BlockSpec((PAIRS, BLK_COLS // 128, 128),
                            lambda j: (0, j, 0))
    return pl.pallas_call(
        _detile_body,
        grid=(grid,),
        in_specs=[in_spec, in_spec],
        out_specs=[out_spec, out_spec],
        out_shape=[out_shape, out_shape],
    )(ut, it)


def _make_sc_kernel(batch):
    info = plsc.get_sparse_core_info()
    nc, ns = info.num_cores, info.num_subcores
    nw = nc * ns
    assert batch % (8 * nw) == 0
    b_per_w = batch // nw
    n_groups = b_per_w // LANES
    mesh = plsc.VectorSubcoreMesh(core_axis_name="c", subcore_axis_name="s")

    @functools.partial(
        pl.kernel,
        mesh=mesh,
        out_type=jax.ShapeDtypeStruct((batch,), jnp.float32),
        scratch_types=[
            pltpu.VMEM((b_per_w,), jnp.int32),
            pltpu.VMEM((b_per_w,), jnp.int32),
            pltpu.VMEM((PAIRS * b_per_w,), jnp.uint32),
            pltpu.VMEM((PAIRS * b_per_w,), jnp.uint32),
            pltpu.VMEM((b_per_w,), jnp.float32),
            pltpu.VMEM((b_per_w,), jnp.float32),
            pltpu.VMEM((LANES,), jnp.float32),
            pltpu.VMEM((b_per_w,), jnp.float32),
            pltpu.SemaphoreType.DMA,
        ],
        compiler_params=pltpu.CompilerParams(needs_layout_passes=False),
    )
    def k(users_hbm, items_hbm, uflat_hbm, iflat_hbm, ubias_hbm, ibias_hbm,
          bias_hbm, out_hbm, users_v, items_v, udat_v, idat_v,
          ubias_v, ibias_v, bias_v, out_v, sem):
        wid = lax.axis_index("s") * nc + lax.axis_index("c")
        base = wid * b_per_w

        pltpu.sync_copy(users_hbm.at[pl.ds(base, b_per_w)], users_v)
        pltpu.sync_copy(items_hbm.at[pl.ds(base, b_per_w)], items_v)
        pltpu.sync_copy(bias_hbm, bias_v)

        copies = []
        for d in range(PAIRS):
            cp = pltpu.make_async_copy(
                uflat_hbm.at[pl.ds(d * DIM_STRIDE, DIM_STRIDE)].at[users_v],
                udat_v.at[pl.ds(d * b_per_w, b_per_w)], sem)
            cp.start()
            copies.append(cp)
            cp = pltpu.make_async_copy(
                iflat_hbm.at[pl.ds(d * DIM_STRIDE, DIM_STRIDE)].at[items_v],
                idat_v.at[pl.ds(d * b_per_w, b_per_w)], sem)
            cp.start()
            copies.append(cp)
        cp_ub = pltpu.make_async_copy(ubias_hbm.at[users_v], ubias_v, sem)
        cp_ib = pltpu.make_async_copy(ibias_hbm.at[items_v], ibias_v, sem)
        cp_ub.start()
        cp_ib.start()
        for cp in copies:
            cp.wait()
        cp_ub.wait()
        cp_ib.wait()

        bias_vec = bias_v[...]
        himask = jnp.full((LANES,), 0xFFFF0000, jnp.uint32)

        def split(w):
            lo = plsc.bitcast(w << 16, jnp.float32)
            hi = plsc.bitcast(w & himask, jnp.float32)
            return lo, hi

        def group(g, carry):
            e0 = g * LANES
            acc = jnp.zeros((LANES,), jnp.float32)
            for d in range(PAIRS):
                sl = pl.ds(d * b_per_w + e0, LANES)
                ulo, uhi = split(udat_v[sl])
                ilo, ihi = split(idat_v[sl])
                acc = acc + ulo * ilo + uhi * ihi
            sl = pl.ds(e0, LANES)
            acc = acc + ubias_v[sl] + ibias_v[sl] + bias_vec
            out_v[sl] = 1.0 / (1.0 + jnp.exp(-acc))
            return carry

        lax.fori_loop(0, n_groups, group, 0)
        pltpu.sync_copy(out_v, out_hbm.at[pl.ds(base, b_per_w)])

    return k


@jax.jit
def kernel(users, items, user_table, item_table, user_bias, item_bias, bias):
    batch = users.shape[0]
    users = users.astype(jnp.int32)
    items = items.astype(jnp.int32)
    u3, i3 = _detile(user_table.T, item_table.T)
    uflat = u3.reshape(-1)
    iflat = i3.reshape(-1)
    bias16 = jnp.broadcast_to(bias.astype(jnp.float32), (LANES,))
    k = _make_sc_kernel(batch)
    return k(users, items, uflat, iflat, user_bias, item_bias, bias16)


# speedup vs baseline: 26.1569x; 1.0428x over previous
import functools

import jax
import jax.numpy as jnp
from jax import lax
from jax.experimental import pallas as pl
from jax.experimental.pallas import tpu as pltpu
from jax.experimental.pallas import tpu_sc as plsc

N_ROWS = 1000000
EMB_DIM = 32
PAIRS = EMB_DIM // 2
LANES = 16
PANELS = 8192
DIM_STRIDE = PANELS * 128
BLK_COLS = 8192


def _pack_block(x):
    bits = jax.lax.bitcast_convert_type(x, jnp.uint32)
    lo = bits[:PAIRS, :] >> 16
    hi = bits[PAIRS:, :] & jnp.uint32(0xFFFF0000)
    packed = lo | hi
    return packed.reshape(PAIRS, BLK_COLS // 128, 128)


def _detile_body(u_in, i_in, u_out, i_out):
    u_out[...] = _pack_block(u_in[...])
    i_out[...] = _pack_block(i_in[...])


def _detile(ut, it):
    grid = (N_ROWS + BLK_COLS - 1) // BLK_COLS
    out_shape = jax.ShapeDtypeStruct((PAIRS, PANELS, 128), jnp.uint32)
    in_spec = pl.BlockSpec((EMB_DIM, BLK_COLS), lambda j: (0, j))
    out_spec = pl.BlockSpec((PAIRS, BLK_COLS // 128, 128),
                            lambda j: (0, j, 0))
    return pl.pallas_call(
        _detile_body,
        grid=(grid,),
        in_specs=[in_spec, in_spec],
        out_specs=[out_spec, out_spec],
        out_shape=[out_shape, out_shape],
    )(ut, it)


def _make_sc_kernel(batch):
    info = plsc.get_sparse_core_info()
    nc, ns = info.num_cores, info.num_subcores
    nw = nc * ns
    assert batch % (8 * nw) == 0
    b_per_w = batch // nw
    n_groups = b_per_w // LANES
    mesh = plsc.VectorSubcoreMesh(core_axis_name="c", subcore_axis_name="s")

    @functools.partial(
        pl.kernel,
        mesh=mesh,
        out_type=jax.ShapeDtypeStruct((batch,), jnp.float32),
        scratch_types=[
            pltpu.VMEM((b_per_w,), jnp.int32),
            pltpu.VMEM((b_per_w,), jnp.int32),
            pltpu.VMEM((PAIRS * b_per_w,), jnp.uint32),
            pltpu.VMEM((PAIRS * b_per_w,), jnp.uint32),
            pltpu.VMEM((b_per_w,), jnp.float32),
            pltpu.VMEM((b_per_w,), jnp.float32),
            pltpu.VMEM((LANES,), jnp.float32),
            pltpu.VMEM((b_per_w,), jnp.float32),
            pltpu.SemaphoreType.DMA,
        ],
        compiler_params=pltpu.CompilerParams(needs_layout_passes=False),
    )
    def k(users_hbm, items_hbm, uflat_hbm, iflat_hbm, ubias_hbm, ibias_hbm,
          bias_hbm, out_hbm, users_v, items_v, udat_v, idat_v,
          ubias_v, ibias_v, bias_v, out_v, sem):
        wid = lax.axis_index("s") * nc + lax.axis_index("c")
        base = wid * b_per_w

        pltpu.sync_copy(users_hbm.at[pl.ds(base, b_per_w)], users_v)
        pltpu.sync_copy(items_hbm.at[pl.ds(base, b_per_w)], items_v)
        pltpu.sync_copy(bias_hbm, bias_v)

        copies = []
        for d in range(PAIRS):
            cp = pltpu.make_async_copy(
                uflat_hbm.at[pl.ds(d * DIM_STRIDE, DIM_STRIDE)].at[users_v],
                udat_v.at[pl.ds(d * b_per_w, b_per_w)], sem)
            cp.start()
            copies.append(cp)
            cp = pltpu.make_async_copy(
                iflat_hbm.at[pl.ds(d * DIM_STRIDE, DIM_STRIDE)].at[items_v],
                idat_v.at[pl.ds(d * b_per_w, b_per_w)], sem)
            cp.start()
            copies.append(cp)
        cp_ub = pltpu.make_async_copy(ubias_hbm.at[users_v], ubias_v, sem)
        cp_ib = pltpu.make_async_copy(ibias_hbm.at[items_v], ibias_v, sem)
        cp_ub.start()
        cp_ib.start()
        for cp in copies:
            cp.wait()
        cp_ub.wait()
        cp_ib.wait()

        bias_vec = bias_v[...]
        himask = jnp.full((LANES,), 0xFFFF0000, jnp.uint32)

        def split(w):
            lo = plsc.bitcast(w << 16, jnp.float32)
            hi = plsc.bitcast(w & himask, jnp.float32)
            return lo, hi

        def group(g, carry):
            e0 = g * LANES
            acc = jnp.zeros((LANES,), jnp.float32)
            for d in range(PAIRS):
                sl = pl.ds(d * b_per_w + e0, LANES)
                ulo, uhi = split(udat_v[sl])
                ilo, ihi = split(idat_v[sl])
                acc = acc + ulo * ilo + uhi * ihi
            sl = pl.ds(e0, LANES)
            acc = acc + ubias_v[sl] + ibias_v[sl] + bias_vec
            out_v[sl] = 1.0 / (1.0 + jnp.exp(-acc))
            return carry

        lax.fori_loop(0, n_groups, group, 0)
        pltpu.sync_copy(out_v, out_hbm.at[pl.ds(base, b_per_w)])

    return k


@jax.jit
def kernel(users, items, user_table, item_table, user_bias, item_bias, bias):
    batch = users.shape[0]
    users = users.astype(jnp.int32)
    items = items.astype(jnp.int32)
    u3, i3 = _detile(user_table.T, item_table.T)
    uflat = u3.reshape(-1)
    iflat = i3.reshape(-1)
    bias16 = jnp.broadcast_to(bias.astype(jnp.float32), (LANES,))
    k = _make_sc_kernel(batch)
    return k(users, items, uflat, iflat, user_bias, item_bias, bias16)


# speedup vs baseline: 30.5841x; 1.1693x over previous
import functools

import jax
import jax.numpy as jnp
from jax import lax
from jax.experimental import pallas as pl
from jax.experimental.pallas import tpu as pltpu
from jax.experimental.pallas import tpu_sc as plsc

N_ROWS = 1000000
EMB_DIM = 32
PAIRS = EMB_DIM // 2
LANES = 16
PANELS = 8192
DIM_STRIDE = PANELS * 128
BLK_COLS = 16384


def _pack_block(x):
    bits = jax.lax.bitcast_convert_type(x, jnp.uint32)
    lo = bits[:PAIRS, :] >> 16
    hi = bits[PAIRS:, :] & jnp.uint32(0xFFFF0000)
    packed = lo | hi
    return packed.reshape(PAIRS, BLK_COLS // 128, 128)


def _detile_body(u_in, i_in, u_out, i_out):
    u_out[...] = _pack_block(u_in[...])
    i_out[...] = _pack_block(i_in[...])


def _detile(ut, it):
    grid = (N_ROWS + BLK_COLS - 1) // BLK_COLS
    out_shape = jax.ShapeDtypeStruct((PAIRS, PANELS, 128), jnp.uint32)
    in_spec = pl.BlockSpec((EMB_DIM, BLK_COLS), lambda j: (0, j))
    out_spec = pl.BlockSpec((PAIRS, BLK_COLS // 128, 128),
                            lambda j: (0, j, 0))
    return pl.pallas_call(
        _detile_body,
        grid=(grid,),
        in_specs=[in_spec, in_spec],
        out_specs=[out_spec, out_spec],
        out_shape=[out_shape, out_shape],
    )(ut, it)


def _make_sc_kernel(batch):
    info = plsc.get_sparse_core_info()
    nc, ns = info.num_cores, info.num_subcores
    nw = nc * ns
    assert batch % (8 * nw) == 0
    b_per_w = batch // nw
    n_groups = b_per_w // LANES
    mesh = plsc.VectorSubcoreMesh(core_axis_name="c", subcore_axis_name="s")

    @functools.partial(
        pl.kernel,
        mesh=mesh,
        out_type=jax.ShapeDtypeStruct((batch,), jnp.float32),
        scratch_types=[
            pltpu.VMEM((b_per_w,), jnp.int32),
            pltpu.VMEM((b_per_w,), jnp.int32),
            pltpu.VMEM((PAIRS * b_per_w,), jnp.uint32),
            pltpu.VMEM((PAIRS * b_per_w,), jnp.uint32),
            pltpu.VMEM((b_per_w,), jnp.float32),
            pltpu.VMEM((b_per_w,), jnp.float32),
            pltpu.VMEM((LANES,), jnp.float32),
            pltpu.VMEM((b_per_w,), jnp.float32),
            pltpu.SemaphoreType.DMA,
        ],
        compiler_params=pltpu.CompilerParams(needs_layout_passes=False),
    )
    def k(users_hbm, items_hbm, uflat_hbm, iflat_hbm, ubias_hbm, ibias_hbm,
          bias_hbm, out_hbm, users_v, items_v, udat_v, idat_v,
          ubias_v, ibias_v, bias_v, out_v, sem):
        wid = lax.axis_index("s") * nc + lax.axis_index("c")
        base = wid * b_per_w

        pltpu.sync_copy(users_hbm.at[pl.ds(base, b_per_w)], users_v)
        pltpu.sync_copy(items_hbm.at[pl.ds(base, b_per_w)], items_v)
        pltpu.sync_copy(bias_hbm, bias_v)

        copies = []
        for d in range(PAIRS):
            cp = pltpu.make_async_copy(
                uflat_hbm.at[pl.ds(d * DIM_STRIDE, DIM_STRIDE)].at[users_v],
                udat_v.at[pl.ds(d * b_per_w, b_per_w)], sem)
            cp.start()
            copies.append(cp)
            cp = pltpu.make_async_copy(
                iflat_hbm.at[pl.ds(d * DIM_STRIDE, DIM_STRIDE)].at[items_v],
                idat_v.at[pl.ds(d * b_per_w, b_per_w)], sem)
            cp.start()
            copies.append(cp)
        cp_ub = pltpu.make_async_copy(ubias_hbm.at[users_v], ubias_v, sem)
        cp_ib = pltpu.make_async_copy(ibias_hbm.at[items_v], ibias_v, sem)
        cp_ub.start()
        cp_ib.start()
        for cp in copies:
            cp.wait()
        cp_ub.wait()
        cp_ib.wait()

        bias_vec = bias_v[...]
        himask = jnp.full((LANES,), 0xFFFF0000, jnp.uint32)

        def split(w):
            lo = plsc.bitcast(w << 16, jnp.float32)
            hi = plsc.bitcast(w & himask, jnp.float32)
            return lo, hi

        def group(g, carry):
            e0 = g * LANES
            acc = jnp.zeros((LANES,), jnp.float32)
            for d in range(PAIRS):
                sl = pl.ds(d * b_per_w + e0, LANES)
                ulo, uhi = split(udat_v[sl])
                ilo, ihi = split(idat_v[sl])
                acc = acc + ulo * ilo + uhi * ihi
            sl = pl.ds(e0, LANES)
            acc = acc + ubias_v[sl] + ibias_v[sl] + bias_vec
            out_v[sl] = 1.0 / (1.0 + jnp.exp(-acc))
            return carry

        lax.fori_loop(0, n_groups, group, 0)
        pltpu.sync_copy(out_v, out_hbm.at[pl.ds(base, b_per_w)])

    return k


@jax.jit
def kernel(users, items, user_table, item_table, user_bias, item_bias, bias):
    batch = users.shape[0]
    users = users.astype(jnp.int32)
    items = items.astype(jnp.int32)
    u3, i3 = _detile(user_table.T, item_table.T)
    uflat = u3.reshape(-1)
    iflat = i3.reshape(-1)
    bias16 = jnp.broadcast_to(bias.astype(jnp.float32), (LANES,))
    k = _make_sc_kernel(batch)
    return k(users, items, uflat, iflat, user_bias, item_bias, bias16)


# speedup vs baseline: 31.6424x; 1.0346x over previous
import functools

import jax
import jax.numpy as jnp
from jax import lax
from jax.experimental import pallas as pl
from jax.experimental.pallas import tpu as pltpu
from jax.experimental.pallas import tpu_sc as plsc

N_ROWS = 1000000
EMB_DIM = 32
PAIRS = EMB_DIM // 2
LANES = 16
PANELS = 8192
DIM_STRIDE = PANELS * 128
BLK_COLS = 32768


def _pack_block(x):
    bits = jax.lax.bitcast_convert_type(x, jnp.uint32)
    lo = bits[:PAIRS, :] >> 16
    hi = bits[PAIRS:, :] & jnp.uint32(0xFFFF0000)
    packed = lo | hi
    return packed.reshape(PAIRS, BLK_COLS // 128, 128)


def _detile_body(u_in, i_in, u_out, i_out):
    u_out[...] = _pack_block(u_in[...])
    i_out[...] = _pack_block(i_in[...])


def _detile(ut, it):
    grid = (N_ROWS + BLK_COLS - 1) // BLK_COLS
    out_shape = jax.ShapeDtypeStruct((PAIRS, PANELS, 128), jnp.uint32)
    in_spec = pl.BlockSpec((EMB_DIM, BLK_COLS), lambda j: (0, j))
    out_spec = pl.BlockSpec((PAIRS, BLK_COLS // 128, 128),
                            lambda j: (0, j, 0))
    return pl.pallas_call(
        _detile_body,
        grid=(grid,),
        in_specs=[in_spec, in_spec],
        out_specs=[out_spec, out_spec],
        out_shape=[out_shape, out_shape],
    )(ut, it)


def _make_sc_kernel(batch):
    info = plsc.get_sparse_core_info()
    nc, ns = info.num_cores, info.num_subcores
    nw = nc * ns
    assert batch % (8 * nw) == 0
    b_per_w = batch // nw
    n_groups = b_per_w // LANES
    mesh = plsc.VectorSubcoreMesh(core_axis_name="c", subcore_axis_name="s")

    @functools.partial(
        pl.kernel,
        mesh=mesh,
        out_type=jax.ShapeDtypeStruct((batch,), jnp.float32),
        scratch_types=[
            pltpu.VMEM((b_per_w,), jnp.int32),
            pltpu.VMEM((b_per_w,), jnp.int32),
            pltpu.VMEM((PAIRS * b_per_w,), jnp.uint32),
            pltpu.VMEM((PAIRS * b_per_w,), jnp.uint32),
            pltpu.VMEM((b_per_w,), jnp.float32),
            pltpu.VMEM((b_per_w,), jnp.float32),
            pltpu.VMEM((LANES,), jnp.float32),
            pltpu.VMEM((b_per_w,), jnp.float32),
            pltpu.SemaphoreType.DMA,
        ],
        compiler_params=pltpu.CompilerParams(needs_layout_passes=False),
    )
    def k(users_hbm, items_hbm, uflat_hbm, iflat_hbm, ubias_hbm, ibias_hbm,
          bias_hbm, out_hbm, users_v, items_v, udat_v, idat_v,
          ubias_v, ibias_v, bias_v, out_v, sem):
        wid = lax.axis_index("s") * nc + lax.axis_index("c")
        base = wid * b_per_w

        pltpu.sync_copy(users_hbm.at[pl.ds(base, b_per_w)], users_v)
        pltpu.sync_copy(items_hbm.at[pl.ds(base, b_per_w)], items_v)
        pltpu.sync_copy(bias_hbm, bias_v)

        copies = []
        for d in range(PAIRS):
            cp = pltpu.make_async_copy(
                uflat_hbm.at[pl.ds(d * DIM_STRIDE, DIM_STRIDE)].at[users_v],
                udat_v.at[pl.ds(d * b_per_w, b_per_w)], sem)
            cp.start()
            copies.append(cp)
            cp = pltpu.make_async_copy(
                iflat_hbm.at[pl.ds(d * DIM_STRIDE, DIM_STRIDE)].at[items_v],
                idat_v.at[pl.ds(d * b_per_w, b_per_w)], sem)
            cp.start()
            copies.append(cp)
        cp_ub = pltpu.make_async_copy(ubias_hbm.at[users_v], ubias_v, sem)
        cp_ib = pltpu.make_async_copy(ibias_hbm.at[items_v], ibias_v, sem)
        cp_ub.start()
        cp_ib.start()
        for cp in copies:
            cp.wait()
        cp_ub.wait()
        cp_ib.wait()

        bias_vec = bias_v[...]
        himask = jnp.full((LANES,), 0xFFFF0000, jnp.uint32)

        def split(w):
            lo = plsc.bitcast(w << 16, jnp.float32)
            hi = plsc.bitcast(w & himask, jnp.float32)
            return lo, hi

        def group(g, carry):
            e0 = g * LANES
            acc = jnp.zeros((LANES,), jnp.float32)
            for d in range(PAIRS):
                sl = pl.ds(d * b_per_w + e0, LANES)
                ulo, uhi = split(udat_v[sl])
                ilo, ihi = split(idat_v[sl])
                acc = acc + ulo * ilo + uhi * ihi
            sl = pl.ds(e0, LANES)
            acc = acc + ubias_v[sl] + ibias_v[sl] + bias_vec
            out_v[sl] = 1.0 / (1.0 + jnp.exp(-acc))
            return carry

        lax.fori_loop(0, n_groups, group, 0)
        pltpu.sync_copy(out_v, out_hbm.at[pl.ds(base, b_per_w)])

    return k


@jax.jit
def kernel(users, items, user_table, item_table, user_bias, item_bias, bias):
    batch = users.shape[0]
    users = users.astype(jnp.int32)
    items = items.astype(jnp.int32)
    u3, i3 = _detile(user_table.T, item_table.T)
    uflat = u3.reshape(-1)
    iflat = i3.reshape(-1)
    bias16 = jnp.broadcast_to(bias.astype(jnp.float32), (LANES,))
    k = _make_sc_kernel(batch)
    return k(users, items, uflat, iflat, user_bias, item_bias, bias16)
